# baseline (speedup 1.0000x reference)
import functools

import jax
import jax.numpy as jnp
from jax import lax
from jax.experimental import pallas as pl
from jax.experimental.pallas import tpu as pltpu
from jax.experimental.pallas import tpu_sc as plsc

NUM_EMBED = 1000000
EMBED_DIM = 32
BATCH = 16384
HIST = 50

NC = 2
NS = 16
NW = NC * NS
BBLK = BATCH // NW
CHUNK = 128
PER_H = BBLK // CHUNK
N_CHUNK = HIST * PER_H
NBUF = 2

SUP_ROWS = NUM_EMBED // 4
TCHUNK = 32
N_TCHUNK = SUP_ROWS // TCHUNK
T_ITERS = (N_TCHUNK + NS - 1) // NS
T_TAIL = SUP_ROWS - N_TCHUNK * TCHUNK

_mesh = plsc.VectorSubcoreMesh(core_axis_name="c", subcore_axis_name="s")


@functools.partial(
    pl.kernel,
    mesh=_mesh,
    out_type=jax.ShapeDtypeStruct((HIST, EMBED_DIM, BATCH), jnp.float32),
    scratch_types=[
        pltpu.HBM((NC, SUP_ROWS, 128), jnp.float32),
        pltpu.VMEM((HIST, BBLK), jnp.int32),
        [pltpu.VMEM((CHUNK, 128), jnp.float32) for _ in range(NBUF)],
        [pltpu.VMEM((EMBED_DIM, CHUNK), jnp.float32) for _ in range(NBUF)],
        [pltpu.VMEM((CHUNK // 128, 128), jnp.int32) for _ in range(NBUF)],
        [pltpu.VMEM((CHUNK,), jnp.int32) for _ in range(NBUF)],
        [pltpu.SemaphoreType.DMA for _ in range(NBUF)],
        [pltpu.SemaphoreType.DMA for _ in range(NBUF)],
    ],
    compiler_params=pltpu.CompilerParams(needs_layout_passes=False),
)
def _gather_kernel(
    wt_hbm, xt_hbm, tail_hbm, out_hbm, tab_hbm, idx_v, gbufs, tbufs,
    sup_idx, qoffs, g_sems, w_sems,
):
    cid = lax.axis_index("c")
    sid = lax.axis_index("s")
    wid = sid * NC + cid
    b0 = wid * BBLK

    pltpu.sync_copy(xt_hbm.at[:, pl.ds(b0, BBLK)], idx_v)

    iota16 = lax.iota(jnp.int32, 16)

    n_mine = jnp.where(sid < N_TCHUNK - (T_ITERS - 1) * NS, T_ITERS,
                       T_ITERS - 1)

    def t_r0(i):
        return (sid + i * NS) * TCHUNK

    def t_read(r0, b, n):
        off = pl.multiple_of(r0 * 4, 128)
        pltpu.async_copy(
            wt_hbm.at[:, pl.ds(off, n * 4)],
            tbufs[b].at[:, pl.ds(0, n * 4)],
            g_sems[b],
        )

    def t_read_wait(r0, b, n):
        off = pl.multiple_of(r0 * 4, 128)
        pltpu.make_async_copy(
            wt_hbm.at[:, pl.ds(off, n * 4)],
            tbufs[b].at[:, pl.ds(0, n * 4)],
            g_sems[b],
        ).wait()

    def t_shuffle(b, n):
        def c_body(c, carry):
            base = iota16 + c * 16
            jj = lax.shift_right_logical(base, 2)
            qq = lax.bitwise_and(base, 3) * 32
            for f0 in range(EMBED_DIM // 16):
                for d in range(16):
                    fvec = f0 * 16 + lax.bitwise_and(iota16 + d, 15)
                    v = plsc.load_gather(tbufs[b], [fvec, base])
                    plsc.store_scatter(gbufs[b], [jj, qq + fvec], v)
            return carry

        lax.fori_loop(0, n * 4 // 16, c_body, 0)

    def t_write(r0, b, n):
        off = pl.multiple_of(r0, 32)
        pltpu.async_copy(
            gbufs[b].at[pl.ds(0, n)],
            tab_hbm.at[cid, pl.ds(off, n)],
            w_sems[b],
        )

    def t_write_wait(r0, b, n):
        off = pl.multiple_of(r0, 32)
        pltpu.make_async_copy(
            gbufs[b].at[pl.ds(0, n)],
            tab_hbm.at[cid, pl.ds(off, n)],
            w_sems[b],
        ).wait()

    for b in range(NBUF):
        t_read(t_r0(b), b, TCHUNK)

    def t_round(p, carry):
        for b in range(NBUF):
            i = p * NBUF + b
            r0 = t_r0(i)

            @pl.when(i < n_mine)
            def _():
                t_read_wait(r0, b, TCHUNK)
                t_shuffle(b, TCHUNK)
                t_write(r0, b, TCHUNK)
                t_write_wait(r0, b, TCHUNK)

            @pl.when(i + NBUF < n_mine)
            def _():
                t_read(t_r0(i + NBUF), b, TCHUNK)
        return carry

    lax.fori_loop(0, (T_ITERS + NBUF - 1) // NBUF, t_round, 0)

    @pl.when(sid == 0)
    def _():
        pltpu.sync_copy(tail_hbm, gbufs[0].at[pl.ds(0, T_TAIL)])
        pltpu.sync_copy(
            gbufs[0].at[pl.ds(0, T_TAIL)],
            tab_hbm.at[cid, pl.ds(N_TCHUNK * TCHUNK, T_TAIL)],
        )

    plsc.subcore_barrier()

    table = tab_hbm.at[cid]

    def compute(j, b):
        h = lax.div(j, PER_H)
        off = lax.rem(j, PER_H) * CHUNK
        for t in range(CHUNK // 16):
            v = idx_v[h, pl.ds(off + t * 16, 16)]
            sup_idx[b][t // 8, pl.ds((t % 8) * 16, 16)] = (
                lax.shift_right_logical(v, 2)
            )
            qoffs[b][pl.ds(t * 16, 16)] = lax.bitwise_and(v, 3) * 32

    def start_g(b):
        for k in range(CHUNK // 128):
            pltpu.async_copy(
                table.at[sup_idx[b].at[k]],
                gbufs[b].at[pl.ds(k * 128, 128)],
                g_sems[b],
            )

    def wait_g(b):
        for k in range(CHUNK // 128):
            pltpu.make_async_copy(
                table.at[sup_idx[b].at[k]],
                gbufs[b].at[pl.ds(k * 128, 128)],
                g_sems[b],
            ).wait()

    def transpose(b):
        def c_body(c, carry):
            rows = iota16 + c * 16
            qv = qoffs[b][pl.ds(c * 16, 16)]
            for f0 in range(EMBED_DIM // 16):
                for d in range(16):
                    fvec = f0 * 16 + lax.bitwise_and(iota16 + d, 15)
                    v = plsc.load_gather(gbufs[b], [rows, qv + fvec])
                    plsc.store_scatter(tbufs[b], [fvec, rows], v)
            return carry

        lax.fori_loop(0, CHUNK // 16, c_body, 0)

    def start_w(j, b):
        h = lax.div(j, PER_H)
        off = lax.rem(j, PER_H) * CHUNK
        pltpu.async_copy(
            tbufs[b], out_hbm.at[h, :, pl.ds(b0 + off, CHUNK)], w_sems[b]
        )

    def wait_w(j, b):
        h = lax.div(j, PER_H)
        off = lax.rem(j, PER_H) * CHUNK
        pltpu.make_async_copy(
            tbufs[b], out_hbm.at[h, :, pl.ds(b0 + off, CHUNK)], w_sems[b]
        ).wait()

    for b in range(NBUF):
        compute(b, b)
        start_g(b)

    for b in range(NBUF):
        wait_g(b)
        transpose(b)
        start_w(b, b)
        compute(b + NBUF, b)
        start_g(b)

    def round_body(p, carry):
        for b in range(NBUF):
            j = p * NBUF + b
            wait_g(b)
            wait_w(j - NBUF, b)
            transpose(b)
            start_w(j, b)
            compute(j + NBUF, b)
            start_g(b)
        return carry

    lax.fori_loop(1, N_CHUNK // NBUF - 1, round_body, 0)

    for b in range(NBUF):
        j = N_CHUNK - NBUF + b
        wait_g(b)
        wait_w(j - NBUF, b)
        transpose(b)
        start_w(j, b)
    for b in range(NBUF):
        wait_w(N_CHUNK - NBUF + b, b)


def kernel(x, weight):
    xt = x.astype(jnp.int32).T
    wt = weight.T
    tail = weight[N_TCHUNK * TCHUNK * 4 :, :].reshape(T_TAIL, 128)
    out_t = _gather_kernel(wt, xt, tail)
    return jnp.transpose(out_t, (2, 0, 1))


# speedup vs baseline: 1.1769x; 1.1769x over previous
import functools

import jax
import jax.numpy as jnp
from jax import lax
from jax.experimental import pallas as pl
from jax.experimental.pallas import tpu as pltpu
from jax.experimental.pallas import tpu_sc as plsc

NUM_EMBED = 1000000
EMBED_DIM = 32
BATCH = 16384
HIST = 50

NC = 2
NS = 16
NW = NC * NS
BBLK = BATCH // NW
CHUNK = 256
N_CHUNK = HIST * (BBLK // CHUNK)
NBUF = 2

SUP_ROWS = NUM_EMBED // 4

_mesh = plsc.VectorSubcoreMesh(core_axis_name="c", subcore_axis_name="s")


@functools.partial(
    pl.kernel,
    mesh=_mesh,
    out_type=jax.ShapeDtypeStruct((HIST, EMBED_DIM, BATCH), jnp.float32),
    scratch_types=[
        pltpu.VMEM((HIST, BBLK), jnp.int32),
        [pltpu.VMEM((CHUNK, 128), jnp.float32) for _ in range(NBUF)],
        [pltpu.VMEM((EMBED_DIM, CHUNK), jnp.float32) for _ in range(NBUF)],
        [pltpu.VMEM((2, 128), jnp.int32) for _ in range(NBUF)],
        [pltpu.VMEM((CHUNK,), jnp.int32) for _ in range(NBUF)],
        [pltpu.SemaphoreType.DMA for _ in range(NBUF)],
        [pltpu.SemaphoreType.DMA for _ in range(NBUF)],
    ],
    compiler_params=pltpu.CompilerParams(needs_layout_passes=False),
)
def _gather_kernel(
    table_hbm, xt_hbm, out_hbm, idx_v, gbufs, tbufs, sup_idx, qoffs,
    g_sems, w_sems,
):
    wid = lax.axis_index("s") * NC + lax.axis_index("c")
    b0 = wid * BBLK

    pltpu.sync_copy(xt_hbm.at[:, pl.ds(b0, BBLK)], idx_v)

    iota16 = lax.iota(jnp.int32, 16)

    def compute(j, b):
        h = lax.div(j, 2)
        off = lax.rem(j, 2) * CHUNK
        for t in range(CHUNK // 16):
            v = idx_v[h, pl.ds(off + t * 16, 16)]
            sup_idx[b][t // 8, pl.ds((t % 8) * 16, 16)] = (
                lax.shift_right_logical(v, 2)
            )
            qoffs[b][pl.ds(t * 16, 16)] = lax.bitwise_and(v, 3) * 32

    def start_g(b):
        for k in range(2):
            pltpu.async_copy(
                table_hbm.at[sup_idx[b].at[k]],
                gbufs[b].at[pl.ds(k * 128, 128)],
                g_sems[b],
            )

    def wait_g(b):
        for k in range(2):
            pltpu.make_async_copy(
                table_hbm.at[sup_idx[b].at[k]],
                gbufs[b].at[pl.ds(k * 128, 128)],
                g_sems[b],
            ).wait()

    def transpose(b):
        def c_body(c, carry):
            rows = iota16 + c * 16
            qv = qoffs[b][pl.ds(c * 16, 16)]
            for f0 in range(EMBED_DIM // 16):
                for d in range(16):
                    fvec = f0 * 16 + lax.bitwise_and(iota16 + d, 15)
                    v = plsc.load_gather(gbufs[b], [rows, qv + fvec])
                    plsc.store_scatter(tbufs[b], [fvec, rows], v)
            return carry

        lax.fori_loop(0, CHUNK // 16, c_body, 0)

    def start_w(j, b):
        h = lax.div(j, 2)
        off = lax.rem(j, 2) * CHUNK
        pltpu.async_copy(
            tbufs[b], out_hbm.at[h, :, pl.ds(b0 + off, CHUNK)], w_sems[b]
        )

    def wait_w(j, b):
        h = lax.div(j, 2)
        off = lax.rem(j, 2) * CHUNK
        pltpu.make_async_copy(
            tbufs[b], out_hbm.at[h, :, pl.ds(b0 + off, CHUNK)], w_sems[b]
        ).wait()

    for b in range(NBUF):
        compute(b, b)
        start_g(b)

    for b in range(NBUF):
        wait_g(b)
        transpose(b)
        start_w(b, b)
        compute(b + NBUF, b)
        start_g(b)

    def round_body(p, carry):
        for b in range(NBUF):
            j = p * NBUF + b
            wait_g(b)
            wait_w(j - NBUF, b)
            transpose(b)
            start_w(j, b)
            compute(j + NBUF, b)
            start_g(b)
        return carry

    lax.fori_loop(1, N_CHUNK // NBUF - 1, round_body, 0)

    for b in range(NBUF):
        j = N_CHUNK - NBUF + b
        wait_g(b)
        wait_w(j - NBUF, b)
        transpose(b)
        start_w(j, b)
    for b in range(NBUF):
        wait_w(N_CHUNK - NBUF + b, b)


def kernel(x, weight):
    xt = x.astype(jnp.int32).T
    table = weight.reshape(SUP_ROWS, 128)
    out_t = _gather_kernel(table, xt)
    return jnp.transpose(out_t, (2, 0, 1))
